# baseline (speedup 1.0000x reference)
import functools

import jax
import jax.numpy as jnp
from jax import lax
from jax.experimental import pallas as pl
from jax.experimental.pallas import tpu as pltpu
from jax.experimental.pallas import tpu_sc as plsc

_N = 10000
_E = 320000
_H = 128
_G = 256
_STEPS = 4

_NC = 2
_NS = 16
_NW = _NC * _NS
_CHUNK = 128
_NCH = 80
_NPH = 2
_MCH = _NCH // _NPH
_QRT = _CHUNK // 4
_EPT = _NCH * _CHUNK
_EPAD = _NW * _EPT
_R = 10112
_RPT = _R // _NS

_BLK = 2000
_NBLK = _N // _BLK


def _gather_chunk(m_hbm, src_v, rows_v, qsems, j, b):
    for q in range(4):
        pltpu.async_copy(m_hbm.at[src_v.at[j, pl.ds(q * _QRT, _QRT)]],
                         rows_v.at[b, pl.ds(q * _QRT, _QRT)], qsems[b][q])


def _wait_chunk(m_hbm, src_v, rows_v, qsems, j, b):
    for q in range(4):
        pltpu.make_async_copy(m_hbm.at[src_v.at[j, pl.ds(q * _QRT, _QRT)]],
                              rows_v.at[b, pl.ds(q * _QRT, _QRT)],
                              qsems[b][q]).wait()


def _sc_scatter_body(m_hbm, zero_hbm, src_hbm, dst_hbm, out_hbm,
                     agg_sh, src_v, dst_v, rows_v,
                     g00, g01, g02, g03, g10, g11, g12, g13, s0, s1, zsem):
    qsems = ((g00, g01, g02, g03), (g10, g11, g12, g13))
    ssems = (s0, s1)
    c = lax.axis_index("c")
    s = lax.axis_index("s")
    wid = c * _NS + s
    zd = pltpu.async_copy(zero_hbm.at[pl.ds(s * _RPT, _RPT)],
                          agg_sh.at[pl.ds(s * _RPT, _RPT)], zsem)

    for p in range(_NPH):
        pltpu.sync_copy(src_hbm.at[wid, pl.ds(p * _MCH, _MCH)], src_v)
        pltpu.sync_copy(dst_hbm.at[wid, pl.ds(p * _MCH, _MCH)], dst_v)
        _gather_chunk(m_hbm, src_v, rows_v, qsems, 0, 0)
        if p == 0:
            zd.wait()
            plsc.subcore_barrier()

        @pl.loop(0, _MCH, step=2)
        def _grp(j0):
            for t in range(2):
                j = j0 + t
                b = t
                nb = 1 - t
                nx = j + 1
                _wait_chunk(m_hbm, src_v, rows_v, qsems, j, b)
                pltpu.async_copy(rows_v.at[b], agg_sh.at[dst_v.at[j]],
                                 ssems[b], add=True)

                @pl.when(nx < _MCH)
                def _():
                    @pl.when(j > 0)
                    def _():
                        pltpu.make_async_copy(rows_v.at[nb],
                                              agg_sh.at[dst_v.at[j - 1]],
                                              ssems[nb]).wait()

                    _gather_chunk(m_hbm, src_v, rows_v, qsems, nx, nb)

        pltpu.make_async_copy(rows_v.at[0], agg_sh.at[dst_v.at[_MCH - 2]],
                              ssems[0]).wait()
        pltpu.make_async_copy(rows_v.at[1], agg_sh.at[dst_v.at[_MCH - 1]],
                              ssems[1]).wait()

    plsc.subcore_barrier()
    pltpu.sync_copy(agg_sh.at[pl.ds(s * _RPT, _RPT)],
                    out_hbm.at[c, pl.ds(s * _RPT, _RPT)])


_sc_scatter = pl.kernel(
    _sc_scatter_body,
    out_type=jax.ShapeDtypeStruct((_NC, _R, _H), jnp.float32),
    mesh=plsc.VectorSubcoreMesh(core_axis_name="c", subcore_axis_name="s"),
    scratch_types=[
        pltpu.VMEM_SHARED((_R, _H), jnp.float32),
        pltpu.VMEM((_MCH, _CHUNK), jnp.int32),
        pltpu.VMEM((_MCH, _CHUNK), jnp.int32),
        pltpu.VMEM((2, _CHUNK, _H), jnp.float32),
    ] + [pltpu.SemaphoreType.DMA] * 11,
)


def _embed_body(x_ref, wemb_ref, w0_ref, h_ref, m_ref):
    h = jnp.maximum(
        jnp.dot(x_ref[...], wemb_ref[...], preferred_element_type=jnp.float32),
        0.0)
    h_ref[...] = h
    m_ref[...] = jnp.dot(h, w0_ref[...], preferred_element_type=jnp.float32)


_embed = pl.pallas_call(
    _embed_body,
    grid=(_NBLK,),
    in_specs=[
        pl.BlockSpec((_BLK, _H), lambda i: (i, 0)),
        pl.BlockSpec((_H, _H), lambda i: (0, 0)),
        pl.BlockSpec((_H, _H), lambda i: (0, 0)),
    ],
    out_specs=[
        pl.BlockSpec((_BLK, _H), lambda i: (i, 0)),
        pl.BlockSpec((_BLK, _H), lambda i: (i, 0)),
    ],
    out_shape=[
        jax.ShapeDtypeStruct((_N, _H), jnp.float32),
        jax.ShapeDtypeStruct((_N, _H), jnp.float32),
    ],
)


def _gru_body(last, p0_ref, p1_ref, h_ref, wih_ref, whh_ref, bih_ref,
              bhh_ref, wn_ref, h_out, m_out=None):
    agg = p0_ref[...] + p1_ref[...]
    gi = jnp.dot(agg, wih_ref[...],
                 preferred_element_type=jnp.float32) + bih_ref[...]
    gh = jnp.dot(h_ref[...], whh_ref[...],
                 preferred_element_type=jnp.float32) + bhh_ref[...]
    r = jax.nn.sigmoid(gi[:, :_H] + gh[:, :_H])
    z = jax.nn.sigmoid(gi[:, _H:2 * _H] + gh[:, _H:2 * _H])
    n = jnp.tanh(gi[:, 2 * _H:] + r * gh[:, 2 * _H:])
    h_new = (1.0 - z) * n + z * h_ref[...]
    if last:
        h_out[...] = jnp.maximum(h_new, 0.0)
    else:
        h_out[...] = h_new
        m_out[...] = jnp.dot(h_new, wn_ref[...],
                             preferred_element_type=jnp.float32)


def _make_gru(last):
    n_out = 1 if last else 2
    return pl.pallas_call(
        functools.partial(_gru_body, last),
        grid=(_NBLK,),
        in_specs=[
            pl.BlockSpec((_BLK, _H), lambda i: (i, 0)),
            pl.BlockSpec((_BLK, _H), lambda i: (i, 0)),
            pl.BlockSpec((_BLK, _H), lambda i: (i, 0)),
            pl.BlockSpec((_H, 3 * _H), lambda i: (0, 0)),
            pl.BlockSpec((_H, 3 * _H), lambda i: (0, 0)),
            pl.BlockSpec((1, 3 * _H), lambda i: (0, 0)),
            pl.BlockSpec((1, 3 * _H), lambda i: (0, 0)),
            pl.BlockSpec((_H, _H), lambda i: (0, 0)),
        ],
        out_specs=[pl.BlockSpec((_BLK, _H), lambda i: (i, 0))] * n_out,
        out_shape=[jax.ShapeDtypeStruct((_N, _H), jnp.float32)] * n_out,
    )


_gru_mid = _make_gru(False)


def _gru_pool_body(p0_ref, p1_ref, h_ref, wih_ref, whh_ref, bih_ref,
                   bhh_ref, batch_ref, w1_ref, b1_ref, w2_ref, b2_ref,
                   out_ref, sums_sc, counts_sc):
    i = pl.program_id(0)

    @pl.when(i == 0)
    def _():
        sums_sc[...] = jnp.zeros_like(sums_sc)
        counts_sc[...] = jnp.zeros_like(counts_sc)

    agg = p0_ref[...] + p1_ref[...]
    gi = jnp.dot(agg, wih_ref[...],
                 preferred_element_type=jnp.float32) + bih_ref[...]
    gh = jnp.dot(h_ref[...], whh_ref[...],
                 preferred_element_type=jnp.float32) + bhh_ref[...]
    r = jax.nn.sigmoid(gi[:, :_H] + gh[:, :_H])
    z = jax.nn.sigmoid(gi[:, _H:2 * _H] + gh[:, _H:2 * _H])
    n = jnp.tanh(gi[:, 2 * _H:] + r * gh[:, 2 * _H:])
    hr = jnp.maximum((1.0 - z) * n + z * h_ref[...], 0.0)

    seg = lax.broadcasted_iota(jnp.int32, (_G, _BLK), 0)
    onehot = (seg == batch_ref[0]).astype(jnp.float32)
    sums_sc[...] += jnp.dot(onehot, hr, preferred_element_type=jnp.float32)
    counts_sc[...] += jnp.sum(onehot, axis=1, keepdims=True)

    @pl.when(i == _NBLK - 1)
    def _():
        pooled = sums_sc[...] / jnp.maximum(counts_sc[...], 1.0)
        y = jnp.maximum(
            jnp.dot(pooled, w1_ref[...],
                    preferred_element_type=jnp.float32) + b1_ref[...], 0.0)
        o = jnp.dot(y, w2_ref[...],
                    preferred_element_type=jnp.float32) + b2_ref[...]
        out_ref[...] = jax.nn.softplus(o)


_gru_pool = pl.pallas_call(
    _gru_pool_body,
    grid=(_NBLK,),
    in_specs=[
        pl.BlockSpec((_BLK, _H), lambda i: (i, 0)),
        pl.BlockSpec((_BLK, _H), lambda i: (i, 0)),
        pl.BlockSpec((_BLK, _H), lambda i: (i, 0)),
        pl.BlockSpec((_H, 3 * _H), lambda i: (0, 0)),
        pl.BlockSpec((_H, 3 * _H), lambda i: (0, 0)),
        pl.BlockSpec((1, 3 * _H), lambda i: (0, 0)),
        pl.BlockSpec((1, 3 * _H), lambda i: (0, 0)),
        pl.BlockSpec((1, 1, _BLK), lambda i: (i, 0, 0)),
        pl.BlockSpec((_H, _H), lambda i: (0, 0)),
        pl.BlockSpec((1, _H), lambda i: (0, 0)),
        pl.BlockSpec((_H, _H), lambda i: (0, 0)),
        pl.BlockSpec((1, _H), lambda i: (0, 0)),
    ],
    out_specs=pl.BlockSpec((_G, _H), lambda i: (0, 0)),
    out_shape=jax.ShapeDtypeStruct((_G, _H), jnp.float32),
    scratch_shapes=[
        pltpu.VMEM((_G, _H), jnp.float32),
        pltpu.VMEM((_G, 1), jnp.float32),
    ],
)


def kernel(x, edge_index, batch, W_emb, ggc_w, w_ih, w_hh, b_ih, b_hh,
           W1, b1, W2, b2):
    src = edge_index[0].astype(jnp.int32)
    dst = edge_index[1].astype(jnp.int32)
    pad = _EPAD - _E
    pad_i = jnp.arange(pad, dtype=jnp.int32)
    src_p = jnp.concatenate([src, pad_i % _N])
    dst_p = jnp.concatenate([dst, _N + pad_i % (_R - _N)])
    src3 = src_p.reshape(_NW, _NCH, _CHUNK)
    dst3 = dst_p.reshape(_NW, _NCH, _CHUNK)
    zeros = jnp.zeros((_R, _H), jnp.float32)

    wih_t = w_ih.T
    whh_t = w_hh.T
    bih2 = b_ih.reshape(1, 3 * _H)
    bhh2 = b_hh.reshape(1, 3 * _H)
    w2p = jnp.pad(W2, ((0, 0), (0, _H - W2.shape[1])))
    b2b = jnp.broadcast_to(b2, (1, _H))
    b12 = b1.reshape(1, _H)
    batch3 = batch.astype(jnp.int32).reshape(_NBLK, 1, _BLK)

    h, m = _embed(x, W_emb, ggc_w[0])
    for i in range(_STEPS - 1):
        partial = _sc_scatter(m, zeros, src3, dst3)
        h, m = _gru_mid(partial[0], partial[1], h, wih_t, whh_t,
                        bih2, bhh2, ggc_w[i + 1])
    partial = _sc_scatter(m, zeros, src3, dst3)
    out = _gru_pool(partial[0], partial[1], h, wih_t, whh_t, bih2, bhh2,
                    batch3, W1, b12, w2p, b2b)
    return out[:, 0]


# speedup vs baseline: 1.0036x; 1.0036x over previous
import functools

import jax
import jax.numpy as jnp
from jax import lax
from jax.experimental import pallas as pl
from jax.experimental.pallas import tpu as pltpu
from jax.experimental.pallas import tpu_sc as plsc

_N = 10000
_E = 320000
_H = 128
_G = 256
_STEPS = 4

_NC = 2
_NS = 16
_NW = _NC * _NS
_CHUNK = 128
_NCH = 80
_NPH = 2
_MCH = _NCH // _NPH
_NSTR = 8
_QRT = _CHUNK // _NSTR
_EPT = _NCH * _CHUNK
_EPAD = _NW * _EPT
_R = 10112
_RPT = _R // _NS

_BLK = 2000
_NBLK = _N // _BLK


def _gather_chunk(m_hbm, src_v, rows_v, qsems, j, b):
    for q in range(_NSTR):
        pltpu.async_copy(m_hbm.at[src_v.at[j, pl.ds(q * _QRT, _QRT)]],
                         rows_v.at[b, pl.ds(q * _QRT, _QRT)], qsems[b][q])


def _wait_chunk(m_hbm, src_v, rows_v, qsems, j, b):
    for q in range(_NSTR):
        pltpu.make_async_copy(m_hbm.at[src_v.at[j, pl.ds(q * _QRT, _QRT)]],
                              rows_v.at[b, pl.ds(q * _QRT, _QRT)],
                              qsems[b][q]).wait()


def _sc_scatter_body(m_hbm, zero_hbm, src_hbm, dst_hbm, out_hbm,
                     agg_sh, src_v, dst_v, rows_v, *sems):
    qsems = (sems[:_NSTR], sems[_NSTR:2 * _NSTR])
    ssems = (sems[2 * _NSTR], sems[2 * _NSTR + 1])
    zsem = sems[2 * _NSTR + 2]
    c = lax.axis_index("c")
    s = lax.axis_index("s")
    wid = c * _NS + s
    zd = pltpu.async_copy(zero_hbm.at[pl.ds(s * _RPT, _RPT)],
                          agg_sh.at[pl.ds(s * _RPT, _RPT)], zsem)

    for p in range(_NPH):
        pltpu.sync_copy(src_hbm.at[wid, pl.ds(p * _MCH, _MCH)], src_v)
        pltpu.sync_copy(dst_hbm.at[wid, pl.ds(p * _MCH, _MCH)], dst_v)
        _gather_chunk(m_hbm, src_v, rows_v, qsems, 0, 0)
        if p == 0:
            zd.wait()
            plsc.subcore_barrier()

        @pl.loop(0, _MCH, step=2)
        def _grp(j0):
            for t in range(2):
                j = j0 + t
                b = t
                nb = 1 - t
                nx = j + 1
                _wait_chunk(m_hbm, src_v, rows_v, qsems, j, b)
                pltpu.async_copy(rows_v.at[b], agg_sh.at[dst_v.at[j]],
                                 ssems[b], add=True)

                @pl.when(nx < _MCH)
                def _():
                    @pl.when(j > 0)
                    def _():
                        pltpu.make_async_copy(rows_v.at[nb],
                                              agg_sh.at[dst_v.at[j - 1]],
                                              ssems[nb]).wait()

                    _gather_chunk(m_hbm, src_v, rows_v, qsems, nx, nb)

        pltpu.make_async_copy(rows_v.at[0], agg_sh.at[dst_v.at[_MCH - 2]],
                              ssems[0]).wait()
        pltpu.make_async_copy(rows_v.at[1], agg_sh.at[dst_v.at[_MCH - 1]],
                              ssems[1]).wait()

    plsc.subcore_barrier()
    pltpu.sync_copy(agg_sh.at[pl.ds(s * _RPT, _RPT)],
                    out_hbm.at[c, pl.ds(s * _RPT, _RPT)])


_sc_scatter = pl.kernel(
    _sc_scatter_body,
    out_type=jax.ShapeDtypeStruct((_NC, _R, _H), jnp.float32),
    mesh=plsc.VectorSubcoreMesh(core_axis_name="c", subcore_axis_name="s"),
    scratch_types=[
        pltpu.VMEM_SHARED((_R, _H), jnp.float32),
        pltpu.VMEM((_MCH, _CHUNK), jnp.int32),
        pltpu.VMEM((_MCH, _CHUNK), jnp.int32),
        pltpu.VMEM((2, _CHUNK, _H), jnp.float32),
    ] + [pltpu.SemaphoreType.DMA] * (2 * _NSTR + 3),
)


def _embed_body(x_ref, wemb_ref, w0_ref, h_ref, m_ref):
    h = jnp.maximum(
        jnp.dot(x_ref[...], wemb_ref[...], preferred_element_type=jnp.float32),
        0.0)
    h_ref[...] = h
    m_ref[...] = jnp.dot(h, w0_ref[...], preferred_element_type=jnp.float32)


_embed = pl.pallas_call(
    _embed_body,
    grid=(_NBLK,),
    in_specs=[
        pl.BlockSpec((_BLK, _H), lambda i: (i, 0)),
        pl.BlockSpec((_H, _H), lambda i: (0, 0)),
        pl.BlockSpec((_H, _H), lambda i: (0, 0)),
    ],
    out_specs=[
        pl.BlockSpec((_BLK, _H), lambda i: (i, 0)),
        pl.BlockSpec((_BLK, _H), lambda i: (i, 0)),
    ],
    out_shape=[
        jax.ShapeDtypeStruct((_N, _H), jnp.float32),
        jax.ShapeDtypeStruct((_N, _H), jnp.float32),
    ],
)


def _gru_body(last, p0_ref, p1_ref, h_ref, wih_ref, whh_ref, bih_ref,
              bhh_ref, wn_ref, h_out, m_out=None):
    agg = p0_ref[...] + p1_ref[...]
    gi = jnp.dot(agg, wih_ref[...],
                 preferred_element_type=jnp.float32) + bih_ref[...]
    gh = jnp.dot(h_ref[...], whh_ref[...],
                 preferred_element_type=jnp.float32) + bhh_ref[...]
    r = jax.nn.sigmoid(gi[:, :_H] + gh[:, :_H])
    z = jax.nn.sigmoid(gi[:, _H:2 * _H] + gh[:, _H:2 * _H])
    n = jnp.tanh(gi[:, 2 * _H:] + r * gh[:, 2 * _H:])
    h_new = (1.0 - z) * n + z * h_ref[...]
    if last:
        h_out[...] = jnp.maximum(h_new, 0.0)
    else:
        h_out[...] = h_new
        m_out[...] = jnp.dot(h_new, wn_ref[...],
                             preferred_element_type=jnp.float32)


def _make_gru(last):
    n_out = 1 if last else 2
    return pl.pallas_call(
        functools.partial(_gru_body, last),
        grid=(_NBLK,),
        in_specs=[
            pl.BlockSpec((_BLK, _H), lambda i: (i, 0)),
            pl.BlockSpec((_BLK, _H), lambda i: (i, 0)),
            pl.BlockSpec((_BLK, _H), lambda i: (i, 0)),
            pl.BlockSpec((_H, 3 * _H), lambda i: (0, 0)),
            pl.BlockSpec((_H, 3 * _H), lambda i: (0, 0)),
            pl.BlockSpec((1, 3 * _H), lambda i: (0, 0)),
            pl.BlockSpec((1, 3 * _H), lambda i: (0, 0)),
            pl.BlockSpec((_H, _H), lambda i: (0, 0)),
        ],
        out_specs=[pl.BlockSpec((_BLK, _H), lambda i: (i, 0))] * n_out,
        out_shape=[jax.ShapeDtypeStruct((_N, _H), jnp.float32)] * n_out,
    )


_gru_mid = _make_gru(False)


def _gru_pool_body(p0_ref, p1_ref, h_ref, wih_ref, whh_ref, bih_ref,
                   bhh_ref, batch_ref, w1_ref, b1_ref, w2_ref, b2_ref,
                   out_ref, sums_sc, counts_sc):
    i = pl.program_id(0)

    @pl.when(i == 0)
    def _():
        sums_sc[...] = jnp.zeros_like(sums_sc)
        counts_sc[...] = jnp.zeros_like(counts_sc)

    agg = p0_ref[...] + p1_ref[...]
    gi = jnp.dot(agg, wih_ref[...],
                 preferred_element_type=jnp.float32) + bih_ref[...]
    gh = jnp.dot(h_ref[...], whh_ref[...],
                 preferred_element_type=jnp.float32) + bhh_ref[...]
    r = jax.nn.sigmoid(gi[:, :_H] + gh[:, :_H])
    z = jax.nn.sigmoid(gi[:, _H:2 * _H] + gh[:, _H:2 * _H])
    n = jnp.tanh(gi[:, 2 * _H:] + r * gh[:, 2 * _H:])
    hr = jnp.maximum((1.0 - z) * n + z * h_ref[...], 0.0)

    seg = lax.broadcasted_iota(jnp.int32, (_G, _BLK), 0)
    onehot = (seg == batch_ref[0]).astype(jnp.float32)
    sums_sc[...] += jnp.dot(onehot, hr, preferred_element_type=jnp.float32)
    counts_sc[...] += jnp.sum(onehot, axis=1, keepdims=True)

    @pl.when(i == _NBLK - 1)
    def _():
        pooled = sums_sc[...] / jnp.maximum(counts_sc[...], 1.0)
        y = jnp.maximum(
            jnp.dot(pooled, w1_ref[...],
                    preferred_element_type=jnp.float32) + b1_ref[...], 0.0)
        o = jnp.dot(y, w2_ref[...],
                    preferred_element_type=jnp.float32) + b2_ref[...]
        out_ref[...] = jax.nn.softplus(o)


_gru_pool = pl.pallas_call(
    _gru_pool_body,
    grid=(_NBLK,),
    in_specs=[
        pl.BlockSpec((_BLK, _H), lambda i: (i, 0)),
        pl.BlockSpec((_BLK, _H), lambda i: (i, 0)),
        pl.BlockSpec((_BLK, _H), lambda i: (i, 0)),
        pl.BlockSpec((_H, 3 * _H), lambda i: (0, 0)),
        pl.BlockSpec((_H, 3 * _H), lambda i: (0, 0)),
        pl.BlockSpec((1, 3 * _H), lambda i: (0, 0)),
        pl.BlockSpec((1, 3 * _H), lambda i: (0, 0)),
        pl.BlockSpec((1, 1, _BLK), lambda i: (i, 0, 0)),
        pl.BlockSpec((_H, _H), lambda i: (0, 0)),
        pl.BlockSpec((1, _H), lambda i: (0, 0)),
        pl.BlockSpec((_H, _H), lambda i: (0, 0)),
        pl.BlockSpec((1, _H), lambda i: (0, 0)),
    ],
    out_specs=pl.BlockSpec((_G, _H), lambda i: (0, 0)),
    out_shape=jax.ShapeDtypeStruct((_G, _H), jnp.float32),
    scratch_shapes=[
        pltpu.VMEM((_G, _H), jnp.float32),
        pltpu.VMEM((_G, 1), jnp.float32),
    ],
)


def kernel(x, edge_index, batch, W_emb, ggc_w, w_ih, w_hh, b_ih, b_hh,
           W1, b1, W2, b2):
    src = edge_index[0].astype(jnp.int32)
    dst = edge_index[1].astype(jnp.int32)
    pad = _EPAD - _E
    pad_i = jnp.arange(pad, dtype=jnp.int32)
    src_p = jnp.concatenate([src, pad_i % _N])
    dst_p = jnp.concatenate([dst, _N + pad_i % (_R - _N)])
    src3 = src_p.reshape(_NW, _NCH, _CHUNK)
    dst3 = dst_p.reshape(_NW, _NCH, _CHUNK)
    zeros = jnp.zeros((_R, _H), jnp.float32)

    wih_t = w_ih.T
    whh_t = w_hh.T
    bih2 = b_ih.reshape(1, 3 * _H)
    bhh2 = b_hh.reshape(1, 3 * _H)
    w2p = jnp.pad(W2, ((0, 0), (0, _H - W2.shape[1])))
    b2b = jnp.broadcast_to(b2, (1, _H))
    b12 = b1.reshape(1, _H)
    batch3 = batch.astype(jnp.int32).reshape(_NBLK, 1, _BLK)

    h, m = _embed(x, W_emb, ggc_w[0])
    for i in range(_STEPS - 1):
        partial = _sc_scatter(m, zeros, src3, dst3)
        h, m = _gru_mid(partial[0], partial[1], h, wih_t, whh_t,
                        bih2, bhh2, ggc_w[i + 1])
    partial = _sc_scatter(m, zeros, src3, dst3)
    out = _gru_pool(partial[0], partial[1], h, wih_t, whh_t, bih2, bhh2,
                    batch3, W1, b12, w2p, b2b)
    return out[:, 0]
